# baseline (speedup 1.0000x reference)
import jax
import jax.numpy as jnp
from jax import lax
from jax.experimental import pallas as pl
from jax.experimental.pallas import tpu as pltpu
from jax.experimental.pallas import tpu_sc as plsc

N = 10000
E = 320000
NC = 2
NS = 16
NW = NC * NS
EPW = E // NW
RPT = 624
TAIL = N - NS * RPT
SROWS = 208
DEG_T = 10
DEG_PER_T = N // DEG_T


def _sc_scatter_make(D, with_deg):
    CH = 400 if D == 64 else 1000
    NCHUNK = EPW // CH
    NPAIR = NCHUNK // 2
    mesh = plsc.VectorSubcoreMesh(
        core_axis_name="c", subcore_axis_name="s", num_cores=NC, num_subcores=NS
    )
    out_type = [jax.ShapeDtypeStruct((NC * N, D), jnp.float32)]
    scratch = [
        pltpu.VMEM_SHARED((N, D), jnp.float32),
        pltpu.VMEM((CH,), jnp.int32),
        pltpu.VMEM((CH,), jnp.int32),
        pltpu.VMEM((CH,), jnp.int32),
        pltpu.VMEM((CH,), jnp.int32),
        pltpu.VMEM((CH, D), jnp.float32),
        pltpu.VMEM((CH, D), jnp.float32),
        pltpu.VMEM((SROWS, D), jnp.float32),
        pltpu.SemaphoreType.DMA,
    ]
    if with_deg:
        out_type.append(jax.ShapeDtypeStruct((NC * N,), jnp.float32))
        scratch += [
            pltpu.VMEM_SHARED((N,), jnp.float32),
            pltpu.VMEM((CH,), jnp.float32),
            pltpu.VMEM((DEG_PER_T,), jnp.float32),
        ]

    def body(y_hbm, src_hbm, dst_hbm, zrows_hbm, zdeg_hbm, out_hbm, *rest):
        if with_deg:
            (deg_hbm, acc, srcv0, dstv0, srcv1, dstv1, rows0, rows1, stage,
             sem, dacc, onesv, dstage) = rest
        else:
            (acc, srcv0, dstv0, srcv1, dstv1, rows0, rows1, stage, sem) = rest
        c = lax.axis_index("c")
        s = lax.axis_index("s")
        wid = s * NC + c
        base = wid * EPW

        rb = s * RPT
        pltpu.sync_copy(zrows_hbm, stage)
        for p in range(RPT // SROWS):
            pltpu.sync_copy(stage, acc.at[pl.ds(rb + p * SROWS, SROWS)])

        @pl.when(s == NS - 1)
        def _():
            pltpu.sync_copy(stage.at[pl.ds(0, TAIL)],
                            acc.at[pl.ds(NS * RPT, TAIL)])

        if with_deg:
            @pl.when(s < DEG_T)
            def _():
                pltpu.sync_copy(zdeg_hbm, dstage)
                pltpu.sync_copy(dstage, dacc.at[pl.ds(s * DEG_PER_T, DEG_PER_T)])

            ones16 = jnp.full((16,), 1.0, jnp.float32)
            for j in range(CH // 16):
                onesv[pl.ds(j * 16, 16)] = ones16
            if CH % 16:
                onesv[pl.ds(CH - 16, 16)] = ones16

        plsc.subcore_barrier()

        pltpu.sync_copy(src_hbm.at[pl.ds(base, CH)], srcv0)
        pltpu.sync_copy(dst_hbm.at[pl.ds(base, CH)], dstv0)
        pltpu.async_copy(y_hbm.at[srcv0], rows0, sem)

        def pair(k, carry):
            i0 = 2 * k
            eb1 = base + (i0 + 1) * CH
            pltpu.sync_copy(src_hbm.at[pl.ds(eb1, CH)], srcv1)
            pltpu.sync_copy(dst_hbm.at[pl.ds(eb1, CH)], dstv1)
            pltpu.make_async_copy(y_hbm.at[srcv0], rows0, sem).wait()
            pltpu.async_copy(y_hbm.at[srcv1], rows1, sem)
            pltpu.sync_copy(rows0, acc.at[dstv0], add=True)
            if with_deg:
                pltpu.sync_copy(onesv, dacc.at[dstv0], add=True)

            @pl.when(i0 + 2 < NCHUNK)
            def _():
                eb2 = base + (i0 + 2) * CH
                pltpu.sync_copy(src_hbm.at[pl.ds(eb2, CH)], srcv0)
                pltpu.sync_copy(dst_hbm.at[pl.ds(eb2, CH)], dstv0)

            pltpu.make_async_copy(y_hbm.at[srcv1], rows1, sem).wait()

            @pl.when(i0 + 2 < NCHUNK)
            def _():
                pltpu.async_copy(y_hbm.at[srcv0], rows0, sem)

            pltpu.sync_copy(rows1, acc.at[dstv1], add=True)
            if with_deg:
                pltpu.sync_copy(onesv, dacc.at[dstv1], add=True)
            return carry

        lax.fori_loop(0, NPAIR, pair, 0)
        if NCHUNK % 2:
            pltpu.make_async_copy(y_hbm.at[srcv0], rows0, sem).wait()
            pltpu.sync_copy(rows0, acc.at[dstv0], add=True)
            if with_deg:
                pltpu.sync_copy(onesv, dacc.at[dstv0], add=True)
        plsc.subcore_barrier()

        ob = c * N + rb
        for p in range(RPT // SROWS):
            pltpu.sync_copy(acc.at[pl.ds(rb + p * SROWS, SROWS)], stage)
            pltpu.sync_copy(stage, out_hbm.at[pl.ds(ob + p * SROWS, SROWS)])

        @pl.when(s == NS - 1)
        def _():
            pltpu.sync_copy(acc.at[pl.ds(NS * RPT, TAIL)],
                            stage.at[pl.ds(0, TAIL)])
            pltpu.sync_copy(stage.at[pl.ds(0, TAIL)],
                            out_hbm.at[pl.ds(c * N + NS * RPT, TAIL)])

        if with_deg:
            @pl.when(s < DEG_T)
            def _():
                pltpu.sync_copy(dacc.at[pl.ds(s * DEG_PER_T, DEG_PER_T)], dstage)
                pltpu.sync_copy(
                    dstage, deg_hbm.at[pl.ds(c * N + s * DEG_PER_T, DEG_PER_T)])

    return pl.kernel(
        body, out_type=out_type, mesh=mesh, scratch_types=scratch,
        compiler_params=pltpu.CompilerParams(use_tc_tiling_on_sc=False),
    )


def _dotT(a, w):
    return lax.dot_general(
        a, w, dimension_numbers=(((1,), (1,)), ((), ())),
        preferred_element_type=jnp.float32,
    )


def _lin_body(x_ref, w_ref, y_ref):
    y_ref[...] = _dotT(x_ref[...], w_ref[...])


def _combine(p_ref, deg_ref, b_ref, xr_ref):
    pv = p_ref[...]
    dv = deg_ref[...]
    deg = jnp.maximum(dv[:N] + dv[N:], 1.0)
    h = (pv[:N] + pv[N:]) / deg + b_ref[...] + xr_ref[...]
    return jnp.maximum(h, 0.0)


def _mid_body(p_ref, deg_ref, b_ref, xr_ref, wl_ref, y_ref, h_ref):
    h = _combine(p_ref, deg_ref, b_ref, xr_ref)
    y_ref[...] = _dotT(h, wl_ref[...])
    h_ref[...] = h


def _out_body(q_ref, deg_ref, b_ref, xr_ref, wc_ref, bc_ref, o_ref):
    h = _combine(q_ref, deg_ref, b_ref, xr_ref)
    o_ref[...] = _dotT(h, wc_ref[...]) + bc_ref[...]


def kernel(x, edge_index, W1_l, b1_l, W1_r, W2_l, b2_l, W2_r, Wc, bc):
    src = edge_index[0].astype(jnp.int32)
    dst = edge_index[1].astype(jnp.int32)
    zrows64 = jnp.zeros((SROWS, 64), jnp.float32)
    zrows32 = jnp.zeros((SROWS, 32), jnp.float32)
    zdeg = jnp.zeros((DEG_PER_T,), jnp.float32)

    f32 = jnp.float32
    y1 = pl.pallas_call(
        _lin_body, out_shape=jax.ShapeDtypeStruct((N, 64), f32),
    )(x, W1_l)

    p, deg = _sc_scatter_make(64, True)(y1, src, dst, zrows64, zdeg)
    xr1 = pl.pallas_call(
        _lin_body, out_shape=jax.ShapeDtypeStruct((N, 64), f32),
    )(x, W1_r)
    deg2 = deg.reshape(NC * N, 1)

    y2, h1 = pl.pallas_call(
        _mid_body,
        out_shape=[jax.ShapeDtypeStruct((N, 32), f32),
                   jax.ShapeDtypeStruct((N, 64), f32)],
    )(p, deg2, b1_l.reshape(1, 64), xr1, W2_l)

    (q,) = _sc_scatter_make(32, False)(y2, src, dst, zrows32, zdeg)
    xr2 = pl.pallas_call(
        _lin_body, out_shape=jax.ShapeDtypeStruct((N, 32), f32),
    )(h1, W2_r)

    out = pl.pallas_call(
        _out_body,
        out_shape=jax.ShapeDtypeStruct((N, 2), f32),
    )(q, deg2, b2_l.reshape(1, 32), xr2, Wc, bc.reshape(1, 2))
    return out


# speedup vs baseline: 1.1557x; 1.1557x over previous
import jax
import jax.numpy as jnp
from jax import lax
from jax.experimental import pallas as pl
from jax.experimental.pallas import tpu as pltpu
from jax.experimental.pallas import tpu_sc as plsc

N = 10000
E = 320000
NC = 2
NS = 16
NW = NC * NS
EPW = E // NW
RPT = 624
TAIL = N - NS * RPT
SROWS = 208
DEG_T = 10
DEG_PER_T = N // DEG_T


def _sc_scatter_make(D, with_deg):
    CH = 400 if D == 64 else 1000
    NCHUNK = EPW // CH
    NPAIR = NCHUNK // 2
    mesh = plsc.VectorSubcoreMesh(
        core_axis_name="c", subcore_axis_name="s", num_cores=NC, num_subcores=NS
    )
    out_type = [jax.ShapeDtypeStruct((N, 128), jnp.float32)]
    scratch = [
        pltpu.VMEM_SHARED((N, D), jnp.float32),
        pltpu.VMEM((CH,), jnp.int32),
        pltpu.VMEM((CH,), jnp.int32),
        pltpu.VMEM((CH,), jnp.int32),
        pltpu.VMEM((CH,), jnp.int32),
        pltpu.VMEM((CH, D), jnp.float32),
        pltpu.VMEM((CH, D), jnp.float32),
        pltpu.VMEM((SROWS, D), jnp.float32),
        pltpu.SemaphoreType.DMA,
    ]
    if with_deg:
        out_type.append(jax.ShapeDtypeStruct((NC * N,), jnp.float32))
        scratch += [
            pltpu.VMEM_SHARED((N,), jnp.float32),
            pltpu.VMEM((CH,), jnp.float32),
            pltpu.VMEM((DEG_PER_T,), jnp.float32),
        ]

    def body(y_hbm, src_hbm, dst_hbm, zrows_hbm, zdeg_hbm, out_hbm, *rest):
        if with_deg:
            (deg_hbm, acc, srcv0, dstv0, srcv1, dstv1, rows0, rows1, stage,
             sem, dacc, onesv, dstage) = rest
        else:
            (acc, srcv0, dstv0, srcv1, dstv1, rows0, rows1, stage, sem) = rest
        c = lax.axis_index("c")
        s = lax.axis_index("s")
        wid = s * NC + c
        base = wid * EPW

        rb = s * RPT
        pltpu.sync_copy(zrows_hbm, stage)
        for p in range(RPT // SROWS):
            pltpu.sync_copy(stage, acc.at[pl.ds(rb + p * SROWS, SROWS)])

        @pl.when(s == NS - 1)
        def _():
            pltpu.sync_copy(stage.at[pl.ds(0, TAIL)],
                            acc.at[pl.ds(NS * RPT, TAIL)])

        if with_deg:
            @pl.when(s < DEG_T)
            def _():
                pltpu.sync_copy(zdeg_hbm, dstage)
                pltpu.sync_copy(dstage, dacc.at[pl.ds(s * DEG_PER_T, DEG_PER_T)])

            ones16 = jnp.full((16,), 1.0, jnp.float32)
            for j in range(CH // 16):
                onesv[pl.ds(j * 16, 16)] = ones16
            if CH % 16:
                onesv[pl.ds(CH - 16, 16)] = ones16

        plsc.subcore_barrier()

        pltpu.sync_copy(src_hbm.at[pl.ds(base, CH)], srcv0)
        pltpu.sync_copy(dst_hbm.at[pl.ds(base, CH)], dstv0)
        pltpu.async_copy(y_hbm.at[srcv0], rows0, sem)

        def pair(k, carry):
            i0 = 2 * k
            eb1 = base + (i0 + 1) * CH
            pltpu.sync_copy(src_hbm.at[pl.ds(eb1, CH)], srcv1)
            pltpu.sync_copy(dst_hbm.at[pl.ds(eb1, CH)], dstv1)
            pltpu.make_async_copy(y_hbm.at[srcv0], rows0, sem).wait()
            pltpu.async_copy(y_hbm.at[srcv1], rows1, sem)
            pltpu.sync_copy(rows0, acc.at[dstv0], add=True)
            if with_deg:
                pltpu.sync_copy(onesv, dacc.at[dstv0], add=True)

            @pl.when(i0 + 2 < NCHUNK)
            def _():
                eb2 = base + (i0 + 2) * CH
                pltpu.sync_copy(src_hbm.at[pl.ds(eb2, CH)], srcv0)
                pltpu.sync_copy(dst_hbm.at[pl.ds(eb2, CH)], dstv0)

            pltpu.make_async_copy(y_hbm.at[srcv1], rows1, sem).wait()

            @pl.when(i0 + 2 < NCHUNK)
            def _():
                pltpu.async_copy(y_hbm.at[srcv0], rows0, sem)

            pltpu.sync_copy(rows1, acc.at[dstv1], add=True)
            if with_deg:
                pltpu.sync_copy(onesv, dacc.at[dstv1], add=True)
            return carry

        lax.fori_loop(0, NPAIR, pair, 0)
        if NCHUNK % 2:
            pltpu.make_async_copy(y_hbm.at[srcv0], rows0, sem).wait()
            pltpu.sync_copy(rows0, acc.at[dstv0], add=True)
            if with_deg:
                pltpu.sync_copy(onesv, dacc.at[dstv0], add=True)
        plsc.subcore_barrier()

        for p in range(RPT // SROWS):
            pltpu.sync_copy(acc.at[pl.ds(rb + p * SROWS, SROWS)], stage)
            pltpu.sync_copy(
                stage, out_hbm.at[pl.ds(rb + p * SROWS, SROWS), pl.ds(D * c, D)])

        @pl.when(s == NS - 1)
        def _():
            pltpu.sync_copy(acc.at[pl.ds(NS * RPT, TAIL)],
                            stage.at[pl.ds(0, TAIL)])
            pltpu.sync_copy(stage.at[pl.ds(0, TAIL)],
                            out_hbm.at[pl.ds(NS * RPT, TAIL), pl.ds(D * c, D)])

        if with_deg:
            @pl.when(s < DEG_T)
            def _():
                pltpu.sync_copy(dacc.at[pl.ds(s * DEG_PER_T, DEG_PER_T)], dstage)
                pltpu.sync_copy(
                    dstage, deg_hbm.at[pl.ds(c * N + s * DEG_PER_T, DEG_PER_T)])

    return pl.kernel(
        body, out_type=out_type, mesh=mesh, scratch_types=scratch,
        compiler_params=pltpu.CompilerParams(use_tc_tiling_on_sc=False),
    )


def _dotT(a, w):
    return lax.dot_general(
        a, w, dimension_numbers=(((1,), (1,)), ((), ())),
        preferred_element_type=jnp.float32,
    )


def _lin2_body(x_ref, wl_ref, wr_ref, y_ref, xr_ref):
    x = x_ref[...]
    y_ref[...] = _dotT(x, wl_ref[...])
    xr_ref[...] = _dotT(x, wr_ref[...])


def _combine(p_ref, deg_ref, b_ref, xr_ref, D):
    pv = p_ref[...]
    psum = pv[:, :D] + pv[:, D:2 * D]
    dv = deg_ref[...]
    deg = jnp.reshape(jnp.maximum(dv[:N] + dv[N:], 1.0), (N, 1))
    h = psum / deg + b_ref[...] + xr_ref[...]
    return jnp.maximum(h, 0.0)


def _mid_body(p_ref, deg_ref, b_ref, xr_ref, wl_ref, wr_ref, y_ref, xr2_ref):
    h = _combine(p_ref, deg_ref, b_ref, xr_ref, 64)
    y_ref[...] = _dotT(h, wl_ref[...])
    xr2_ref[...] = _dotT(h, wr_ref[...])


def _out_body(q_ref, deg_ref, b_ref, xr_ref, wc_ref, bc_ref, o_ref):
    h = _combine(q_ref, deg_ref, b_ref, xr_ref, 32)
    o_ref[...] = _dotT(h, wc_ref[...]) + bc_ref[...]


def kernel(x, edge_index, W1_l, b1_l, W1_r, W2_l, b2_l, W2_r, Wc, bc):
    src = edge_index[0].astype(jnp.int32)
    dst = edge_index[1].astype(jnp.int32)
    zrows64 = jnp.zeros((SROWS, 64), jnp.float32)
    zrows32 = jnp.zeros((SROWS, 32), jnp.float32)
    zdeg = jnp.zeros((DEG_PER_T,), jnp.float32)

    f32 = jnp.float32
    y1, xr1 = pl.pallas_call(
        _lin2_body,
        out_shape=[jax.ShapeDtypeStruct((N, 64), f32),
                   jax.ShapeDtypeStruct((N, 64), f32)],
    )(x, W1_l, W1_r)

    p, deg = _sc_scatter_make(64, True)(y1, src, dst, zrows64, zdeg)

    y2, xr2 = pl.pallas_call(
        _mid_body,
        out_shape=[jax.ShapeDtypeStruct((N, 32), f32),
                   jax.ShapeDtypeStruct((N, 32), f32)],
    )(p, deg, b1_l.reshape(1, 64), xr1, W2_l, W2_r)

    (q,) = _sc_scatter_make(32, False)(y2, src, dst, zrows32, zdeg)

    out = pl.pallas_call(
        _out_body,
        out_shape=jax.ShapeDtypeStruct((N, 2), f32),
    )(q, deg, b2_l.reshape(1, 32), xr2, Wc, bc.reshape(1, 2))
    return out
